# baseline (speedup 1.0000x reference)
import jax
import jax.numpy as jnp
from jax import lax
from jax.experimental import pallas as pl
from jax.experimental.pallas import tpu as pltpu
from jax.experimental.pallas import tpu_sc as plsc

BATCH = 16384
DIM = 1024
GROUP = DIM // 4
ROW_BLOCK = 2048
_LANES = 16
_CHUNK = 32

_LOG2E = 1.4426950408889634
_LN2 = 0.6931471805599453


def _labels_sc_body(ids0_hbm, ids1_hbm, ids2_hbm, ids3_hbm, labels_hbm,
                    idx_v, val_v, sem):
    w = lax.axis_index("s") * 2 + lax.axis_index("c")
    g = w // 8
    chunk = w % 8

    for gg, ids_hbm in enumerate((ids0_hbm, ids1_hbm, ids2_hbm, ids3_hbm)):
        @pl.when(g == gg)
        def _(ids_hbm=ids_hbm):
            pltpu.sync_copy(ids_hbm.at[pl.ds(chunk * _CHUNK, _CHUNK)], idx_v)
    vec = jnp.zeros((_LANES,), jnp.int32) + g
    for i in range(_CHUNK // _LANES):
        val_v[pl.ds(i * _LANES, _LANES)] = vec
    pltpu.async_copy(val_v, labels_hbm.at[idx_v], sem).wait()


_labels_sc = pl.kernel(
    _labels_sc_body,
    out_type=jax.ShapeDtypeStruct((DIM,), jnp.int32),
    mesh=plsc.VectorSubcoreMesh(core_axis_name="c", subcore_axis_name="s"),
    scratch_types=[
        pltpu.VMEM((_CHUNK,), jnp.int32),
        pltpu.VMEM((_CHUNK,), jnp.int32),
        pltpu.SemaphoreType.DMA,
    ],
)


def _link_tc_body(lab_ref, x_ref, o_ref):
    lab = lab_ref[...]
    x = x_ref[...]
    t = jnp.minimum(x * _LOG2E, 115.0)
    z = jnp.exp2(t)
    u = 1.0 + z
    o_ref[...] = jnp.where(
        lab == 0, x,
        jnp.where(lab == 1, z,
                  jnp.where(lab == 2, z / u, _LN2 * jnp.log2(u))))


_link_tc = pl.pallas_call(
    _link_tc_body,
    grid=(BATCH // ROW_BLOCK,),
    in_specs=[pl.BlockSpec((1, DIM), lambda i: (0, 0)),
              pl.BlockSpec((ROW_BLOCK, DIM), lambda i: (i, 0))],
    out_specs=pl.BlockSpec((ROW_BLOCK, DIM), lambda i: (i, 0)),
    out_shape=jax.ShapeDtypeStruct((BATCH, DIM), jnp.float32),
)


def kernel(x, ids0, ids1, ids2, ids3):
    labels = _labels_sc(
        ids0.astype(jnp.int32), ids1.astype(jnp.int32),
        ids2.astype(jnp.int32), ids3.astype(jnp.int32)
    )
    return _link_tc(labels.reshape(1, DIM), x)


# speedup vs baseline: 1.0108x; 1.0108x over previous
import jax
import jax.numpy as jnp
from jax import lax
from jax.experimental import pallas as pl
from jax.experimental.pallas import tpu as pltpu
from jax.experimental.pallas import tpu_sc as plsc

BATCH = 16384
DIM = 1024
GROUP = DIM // 4
ROW_BLOCK = 2048
_LANES = 16
_CHUNK = 64

_LOG2E = 1.4426950408889634
_LN2 = 0.6931471805599453


def _labels_sc_body(ids0_hbm, ids1_hbm, ids2_hbm, ids3_hbm, labels_hbm,
                    idx_v, val_v, sem):
    w = lax.axis_index("s")
    g = w // 4
    chunk = w % 4

    for gg, ids_hbm in enumerate((ids0_hbm, ids1_hbm, ids2_hbm, ids3_hbm)):
        @pl.when(g == gg)
        def _(ids_hbm=ids_hbm):
            pltpu.sync_copy(ids_hbm.at[pl.ds(chunk * _CHUNK, _CHUNK)], idx_v)
    vec = jnp.zeros((_LANES,), jnp.int32) + g
    for i in range(_CHUNK // _LANES):
        val_v[pl.ds(i * _LANES, _LANES)] = vec
    pltpu.async_copy(val_v, labels_hbm.at[idx_v], sem).wait()


_labels_sc = pl.kernel(
    _labels_sc_body,
    out_type=jax.ShapeDtypeStruct((DIM,), jnp.int32),
    mesh=plsc.VectorSubcoreMesh(core_axis_name="c", subcore_axis_name="s",
                                num_cores=1),
    scratch_types=[
        pltpu.VMEM((_CHUNK,), jnp.int32),
        pltpu.VMEM((_CHUNK,), jnp.int32),
        pltpu.SemaphoreType.DMA,
    ],
)


def _link_tc_body(lab_ref, x_ref, o_ref):
    lab = lab_ref[...]
    x = x_ref[...]
    t = jnp.minimum(x * _LOG2E, 115.0)
    z = jnp.exp2(t)
    u = 1.0 + z
    o_ref[...] = jnp.where(
        lab == 0, x,
        jnp.where(lab == 1, z,
                  jnp.where(lab == 2, z / u, _LN2 * jnp.log2(u))))


_link_tc = pl.pallas_call(
    _link_tc_body,
    grid=(BATCH // ROW_BLOCK,),
    in_specs=[pl.BlockSpec((1, DIM), lambda i: (0, 0)),
              pl.BlockSpec((ROW_BLOCK, DIM), lambda i: (i, 0))],
    out_specs=pl.BlockSpec((ROW_BLOCK, DIM), lambda i: (i, 0)),
    out_shape=jax.ShapeDtypeStruct((BATCH, DIM), jnp.float32),
)


def kernel(x, ids0, ids1, ids2, ids3):
    labels = _labels_sc(
        ids0.astype(jnp.int32), ids1.astype(jnp.int32),
        ids2.astype(jnp.int32), ids3.astype(jnp.int32)
    )
    return _link_tc(labels.reshape(1, DIM), x)
